# baseline (speedup 1.0000x reference)
import jax
import jax.numpy as jnp
from jax import lax
from jax.experimental import pallas as pl
from jax.experimental.pallas import tpu as pltpu
from jax.experimental.pallas import tpu_sc as plsc

N_NODES = 10000
N_EDGES = 320000
N_GRAPHS = 64

_NC = 2
_NS = 16
_EDGES_PER_TILE = N_EDGES // (_NC * _NS)
_ROW_CHUNK = 1000
_NROWT = N_NODES // _ROW_CHUNK


def _segsum_sc(z, src, dst, zeros):
    n, d = z.shape
    small_table = d <= 64
    ch = {128: 80, 64: 80, 32: 400}[d]
    nchunks = _EDGES_PER_TILE // ch
    nw = _NC * _NS
    src3 = src.reshape(nw, nchunks, ch)
    dst3 = dst.reshape(nw, nchunks, ch)

    def body(z_hbm, src_hbm, dst_hbm, zero_hbm, out_hbm,
             agg_sh, src_v, dst_v, rows0, rows1, sem0, sem1, *maybe_table):
        c = lax.axis_index("c")
        s = lax.axis_index("s")
        r0 = s * _ROW_CHUNK
        wid = c * _NS + s
        table = maybe_table[0] if small_table else z_hbm

        pltpu.sync_copy(src_hbm.at[wid], src_v)
        pltpu.sync_copy(dst_hbm.at[wid], dst_v)

        @pl.when(s < _NROWT)
        def _():
            pltpu.sync_copy(zero_hbm.at[pl.ds(r0, _ROW_CHUNK)],
                            agg_sh.at[pl.ds(r0, _ROW_CHUNK)])
            if small_table:
                pltpu.sync_copy(z_hbm.at[pl.ds(r0, _ROW_CHUNK)],
                                maybe_table[0].at[pl.ds(r0, _ROW_CHUNK)])
        plsc.subcore_barrier()

        pltpu.async_copy(table.at[src_v.at[0]], rows0, sem0)

        def pair(j, carry):
            i = 2 * j

            @pl.when(i + 1 < nchunks)
            def _():
                pltpu.async_copy(table.at[src_v.at[i + 1]], rows1, sem1)

            pltpu.make_async_copy(z_hbm.at[src_v.at[i]], rows0, sem0).wait()
            pltpu.sync_copy(rows0, agg_sh.at[dst_v.at[i]], add=True)

            @pl.when(i + 2 < nchunks)
            def _():
                pltpu.async_copy(table.at[src_v.at[i + 2]], rows0, sem0)

            @pl.when(i + 1 < nchunks)
            def _():
                pltpu.make_async_copy(z_hbm.at[src_v.at[i + 1]],
                                      rows1, sem1).wait()
                pltpu.sync_copy(rows1, agg_sh.at[dst_v.at[i + 1]], add=True)

            return carry

        lax.fori_loop(0, (nchunks + 1) // 2, pair, 0)

        plsc.subcore_barrier()

        @pl.when(s < _NROWT)
        def _():
            pltpu.sync_copy(agg_sh.at[pl.ds(r0, _ROW_CHUNK)],
                            out_hbm.at[c, pl.ds(r0, _ROW_CHUNK)])

    scratch = [
        pltpu.VMEM_SHARED((n, d), jnp.float32),
        pltpu.VMEM((nchunks, ch), jnp.int32),
        pltpu.VMEM((nchunks, ch), jnp.int32),
        pltpu.VMEM((ch, d), jnp.float32),
        pltpu.VMEM((ch, d), jnp.float32),
        pltpu.SemaphoreType.DMA,
        pltpu.SemaphoreType.DMA,
    ]
    if small_table:
        scratch.append(pltpu.VMEM_SHARED((n, d), jnp.float32))

    return pl.kernel(
        body,
        out_type=jax.ShapeDtypeStruct((_NC, n, d), jnp.float32),
        mesh=plsc.VectorSubcoreMesh(core_axis_name="c", subcore_axis_name="s"),
        compiler_params=pltpu.CompilerParams(use_tc_tiling_on_sc=False),
        scratch_types=scratch,
    )(z, src3, dst3, zeros)


_BLK = 1000
_NBLK = N_NODES // _BLK


def _elu(t):
    return jnp.where(t > 0, t, jnp.exp(jnp.minimum(t, 0.0)) - 1.0)


def _bdot(a, b):
    return jnp.dot(a.astype(jnp.bfloat16), b.astype(jnp.bfloat16),
                   preferred_element_type=jnp.float32)


def _gin_body(h_ref, a0_ref, a1_ref, w_ref, b_ref, o_ref):
    t = _bdot(h_ref[...] + a0_ref[...] + a1_ref[...], w_ref[...]) + b_ref[...]
    o_ref[...] = _elu(t)


def _gin_layer(h, a0, a1, w, b):
    m, k = h.shape
    kk, nn = w.shape
    return pl.pallas_call(
        _gin_body,
        grid=(_NBLK,),
        in_specs=[pl.BlockSpec((_BLK, k), lambda i: (i, 0)),
                  pl.BlockSpec((_BLK, k), lambda i: (i, 0)),
                  pl.BlockSpec((_BLK, k), lambda i: (i, 0)),
                  pl.BlockSpec((k, nn), lambda i: (0, 0)),
                  pl.BlockSpec((1, nn), lambda i: (0, 0))],
        out_specs=pl.BlockSpec((_BLK, nn), lambda i: (i, 0)),
        out_shape=jax.ShapeDtypeStruct((m, nn), jnp.float32),
    )(h, a0, a1, w, b)


def _final_body(h_ref, a0_ref, a1_ref, w3_ref, b3_ref, batch_ref,
                fw1_ref, fb1_ref, fw2_ref, fb2_ref, fw3_ref, fb3_ref,
                o_ref, acc_ref):
    i = pl.program_id(0)
    t = _bdot(h_ref[...] + a0_ref[...] + a1_ref[...], w3_ref[...]) + b3_ref[...]
    h3 = _elu(t)
    gid = batch_ref[...]
    onehot = (gid == lax.broadcasted_iota(jnp.int32, (_BLK, N_GRAPHS), 1)
              ).astype(jnp.float32)
    hb = jnp.concatenate(
        [h3, jnp.ones((_BLK, 1), jnp.float32),
         jnp.zeros((_BLK, 63), jnp.float32)], axis=1)
    part = lax.dot_general(onehot, hb, (((0,), (0,)), ((), ())),
                           preferred_element_type=jnp.float32,
                           precision=lax.Precision.HIGHEST)

    @pl.when(i == 0)
    def _():
        acc_ref[...] = jnp.zeros_like(acc_ref)

    acc_ref[...] += part

    @pl.when(i == _NBLK - 1)
    def _():
        acc = acc_ref[...]
        sums = acc[:, :N_GRAPHS]
        counts = acc[:, N_GRAPHS:N_GRAPHS + 1]
        g = sums / jnp.maximum(counts, 1.0)
        g1 = _elu(_bdot(g, fw1_ref[...]) + fb1_ref[...])
        g2 = _elu(_bdot(g1, fw2_ref[...]) + fb2_ref[...])
        out = lax.dot_general(fw3_ref[...].astype(jnp.bfloat16),
                              g2.astype(jnp.bfloat16),
                              (((0,), (1,)), ((), ())),
                              preferred_element_type=jnp.float32)
        o_ref[...] = out + fb3_ref[...]


def _final_stage(h2, a0, a1, w3, b3, batch2d, fw1, fb1, fw2, fb2, fw3, fb3):
    return pl.pallas_call(
        _final_body,
        grid=(_NBLK,),
        in_specs=[pl.BlockSpec((_BLK, 64), lambda i: (i, 0)),
                  pl.BlockSpec((_BLK, 64), lambda i: (i, 0)),
                  pl.BlockSpec((_BLK, 64), lambda i: (i, 0)),
                  pl.BlockSpec((64, 64), lambda i: (0, 0)),
                  pl.BlockSpec((1, 64), lambda i: (0, 0)),
                  pl.BlockSpec((_BLK, 1), lambda i: (i, 0)),
                  pl.BlockSpec((64, 64), lambda i: (0, 0)),
                  pl.BlockSpec((1, 64), lambda i: (0, 0)),
                  pl.BlockSpec((64, 32), lambda i: (0, 0)),
                  pl.BlockSpec((1, 32), lambda i: (0, 0)),
                  pl.BlockSpec((32, 1), lambda i: (0, 0)),
                  pl.BlockSpec((1, 1), lambda i: (0, 0))],
        out_specs=pl.BlockSpec((1, 64), lambda i: (0, 0)),
        out_shape=jax.ShapeDtypeStruct((1, 64), jnp.float32),
        scratch_shapes=[pltpu.VMEM((64, 128), jnp.float32)],
    )(h2, a0, a1, w3, b3, batch2d, fw1, fb1, fw2, fb2, fw3, fb3)


def kernel(x, edge_index, batch, W1, b1, W2, b2, W3, b3,
           fW1, fb1, fW2, fb2, fW3, fb3):
    src = edge_index[0].astype(jnp.int32)
    dst = edge_index[1].astype(jnp.int32)
    batch2d = batch.astype(jnp.int32).reshape(N_NODES, 1)
    z32 = jnp.zeros((N_NODES, 32), jnp.float32)
    z64 = jnp.zeros((N_NODES, 64), jnp.float32)
    z128 = jnp.zeros((N_NODES, 128), jnp.float32)

    p1 = _segsum_sc(x, src, dst, z128)
    h1 = _gin_layer(x, p1[0], p1[1], W1, b1.reshape(1, -1))

    p2 = _segsum_sc(h1, src, dst, z32)
    h2 = _gin_layer(h1, p2[0], p2[1], W2, b2.reshape(1, -1))

    p3 = _segsum_sc(h2, src, dst, z64)
    out = _final_stage(h2, p3[0], p3[1], W3, b3.reshape(1, -1), batch2d,
                       fW1, fb1.reshape(1, -1), fW2, fb2.reshape(1, -1),
                       fW3, fb3.reshape(1, -1))
    return out.reshape(-1)


# speedup vs baseline: 1.0865x; 1.0865x over previous
import jax
import jax.numpy as jnp
from jax import lax
from jax.experimental import pallas as pl
from jax.experimental.pallas import tpu as pltpu
from jax.experimental.pallas import tpu_sc as plsc

N_NODES = 10000
N_EDGES = 320000
N_GRAPHS = 64

_NC = 2
_NS = 16
_EDGES_PER_TILE = N_EDGES // (_NC * _NS)
_ROW_CHUNK = 1000
_NROWT = N_NODES // _ROW_CHUNK


def _segsum_sc(z, src, dst, zeros):
    n, d = z.shape
    ch = {128: 80, 64: 400, 32: 400}[d]
    nchunks = _EDGES_PER_TILE // ch
    nw = _NC * _NS
    src3 = src.reshape(nw, nchunks, ch)
    dst3 = dst.reshape(nw, nchunks, ch)

    def body(z_hbm, src_hbm, dst_hbm, zero_hbm, out_hbm,
             agg_sh, src_v, dst_v, rows0, rows1, sem0, sem1):
        c = lax.axis_index("c")
        s = lax.axis_index("s")
        r0 = s * _ROW_CHUNK
        wid = c * _NS + s

        pltpu.sync_copy(src_hbm.at[wid], src_v)
        pltpu.sync_copy(dst_hbm.at[wid], dst_v)

        @pl.when(s < _NROWT)
        def _():
            pltpu.sync_copy(zero_hbm.at[pl.ds(r0, _ROW_CHUNK)],
                            agg_sh.at[pl.ds(r0, _ROW_CHUNK)])
        plsc.subcore_barrier()

        pltpu.async_copy(z_hbm.at[src_v.at[0]], rows0, sem0)

        def pair(j, carry):
            i = 2 * j

            @pl.when(i + 1 < nchunks)
            def _():
                pltpu.async_copy(z_hbm.at[src_v.at[i + 1]], rows1, sem1)

            pltpu.make_async_copy(z_hbm.at[src_v.at[i]], rows0, sem0).wait()
            pltpu.sync_copy(rows0, agg_sh.at[dst_v.at[i]], add=True)

            @pl.when(i + 2 < nchunks)
            def _():
                pltpu.async_copy(z_hbm.at[src_v.at[i + 2]], rows0, sem0)

            @pl.when(i + 1 < nchunks)
            def _():
                pltpu.make_async_copy(z_hbm.at[src_v.at[i + 1]],
                                      rows1, sem1).wait()
                pltpu.sync_copy(rows1, agg_sh.at[dst_v.at[i + 1]], add=True)

            return carry

        lax.fori_loop(0, (nchunks + 1) // 2, pair, 0)

        plsc.subcore_barrier()

        @pl.when(s < _NROWT)
        def _():
            pltpu.sync_copy(agg_sh.at[pl.ds(r0, _ROW_CHUNK)],
                            out_hbm.at[c, pl.ds(r0, _ROW_CHUNK)])

    return pl.kernel(
        body,
        out_type=jax.ShapeDtypeStruct((_NC, n, d), jnp.float32),
        mesh=plsc.VectorSubcoreMesh(core_axis_name="c", subcore_axis_name="s"),
        compiler_params=pltpu.CompilerParams(use_tc_tiling_on_sc=False),
        scratch_types=[
            pltpu.VMEM_SHARED((n, d), jnp.float32),
            pltpu.VMEM((nchunks, ch), jnp.int32),
            pltpu.VMEM((nchunks, ch), jnp.int32),
            pltpu.VMEM((ch, d), jnp.float32),
            pltpu.VMEM((ch, d), jnp.float32),
            pltpu.SemaphoreType.DMA,
            pltpu.SemaphoreType.DMA,
        ],
    )(z, src3, dst3, zeros)


_BLK = 1000
_NBLK = N_NODES // _BLK


def _elu(t):
    return jnp.where(t > 0, t, jnp.exp(jnp.minimum(t, 0.0)) - 1.0)


def _bdot(a, b):
    return jnp.dot(a.astype(jnp.bfloat16), b.astype(jnp.bfloat16),
                   preferred_element_type=jnp.float32)


def _gin_body(h_ref, a0_ref, a1_ref, w_ref, b_ref, o_ref):
    t = _bdot(h_ref[...] + a0_ref[...] + a1_ref[...], w_ref[...]) + b_ref[...]
    o_ref[...] = _elu(t)


def _gin_layer(h, a0, a1, w, b):
    m, k = h.shape
    kk, nn = w.shape
    return pl.pallas_call(
        _gin_body,
        grid=(_NBLK,),
        in_specs=[pl.BlockSpec((_BLK, k), lambda i: (i, 0)),
                  pl.BlockSpec((_BLK, k), lambda i: (i, 0)),
                  pl.BlockSpec((_BLK, k), lambda i: (i, 0)),
                  pl.BlockSpec((k, nn), lambda i: (0, 0)),
                  pl.BlockSpec((1, nn), lambda i: (0, 0))],
        out_specs=pl.BlockSpec((_BLK, nn), lambda i: (i, 0)),
        out_shape=jax.ShapeDtypeStruct((m, nn), jnp.float32),
    )(h, a0, a1, w, b)


def _final_body(h_ref, a0_ref, a1_ref, w3_ref, b3_ref, batch_ref,
                fw1_ref, fb1_ref, fw2_ref, fb2_ref, fw3_ref, fb3_ref,
                o_ref, acc_ref):
    i = pl.program_id(0)
    t = _bdot(h_ref[...] + a0_ref[...] + a1_ref[...], w3_ref[...]) + b3_ref[...]
    h3 = _elu(t)
    gid = batch_ref[...]
    onehot = (gid == lax.broadcasted_iota(jnp.int32, (_BLK, N_GRAPHS), 1)
              ).astype(jnp.float32)
    hb = jnp.concatenate(
        [h3, jnp.ones((_BLK, 1), jnp.float32),
         jnp.zeros((_BLK, 63), jnp.float32)], axis=1)
    part = lax.dot_general(onehot, hb, (((0,), (0,)), ((), ())),
                           preferred_element_type=jnp.float32,
                           precision=lax.Precision.HIGHEST)

    @pl.when(i == 0)
    def _():
        acc_ref[...] = jnp.zeros_like(acc_ref)

    acc_ref[...] += part

    @pl.when(i == _NBLK - 1)
    def _():
        acc = acc_ref[...]
        sums = acc[:, :N_GRAPHS]
        counts = acc[:, N_GRAPHS:N_GRAPHS + 1]
        g = sums / jnp.maximum(counts, 1.0)
        g1 = _elu(_bdot(g, fw1_ref[...]) + fb1_ref[...])
        g2 = _elu(_bdot(g1, fw2_ref[...]) + fb2_ref[...])
        out = lax.dot_general(fw3_ref[...].astype(jnp.bfloat16),
                              g2.astype(jnp.bfloat16),
                              (((0,), (1,)), ((), ())),
                              preferred_element_type=jnp.float32)
        o_ref[...] = out + fb3_ref[...]


def _final_stage(h2, a0, a1, w3, b3, batch2d, fw1, fb1, fw2, fb2, fw3, fb3):
    return pl.pallas_call(
        _final_body,
        grid=(_NBLK,),
        in_specs=[pl.BlockSpec((_BLK, 64), lambda i: (i, 0)),
                  pl.BlockSpec((_BLK, 64), lambda i: (i, 0)),
                  pl.BlockSpec((_BLK, 64), lambda i: (i, 0)),
                  pl.BlockSpec((64, 64), lambda i: (0, 0)),
                  pl.BlockSpec((1, 64), lambda i: (0, 0)),
                  pl.BlockSpec((_BLK, 1), lambda i: (i, 0)),
                  pl.BlockSpec((64, 64), lambda i: (0, 0)),
                  pl.BlockSpec((1, 64), lambda i: (0, 0)),
                  pl.BlockSpec((64, 32), lambda i: (0, 0)),
                  pl.BlockSpec((1, 32), lambda i: (0, 0)),
                  pl.BlockSpec((32, 1), lambda i: (0, 0)),
                  pl.BlockSpec((1, 1), lambda i: (0, 0))],
        out_specs=pl.BlockSpec((1, 64), lambda i: (0, 0)),
        out_shape=jax.ShapeDtypeStruct((1, 64), jnp.float32),
        scratch_shapes=[pltpu.VMEM((64, 128), jnp.float32)],
    )(h2, a0, a1, w3, b3, batch2d, fw1, fb1, fw2, fb2, fw3, fb3)


def kernel(x, edge_index, batch, W1, b1, W2, b2, W3, b3,
           fW1, fb1, fW2, fb2, fW3, fb3):
    src = edge_index[0].astype(jnp.int32)
    dst = edge_index[1].astype(jnp.int32)
    batch2d = batch.astype(jnp.int32).reshape(N_NODES, 1)
    z32 = jnp.zeros((N_NODES, 32), jnp.float32)
    z64 = jnp.zeros((N_NODES, 64), jnp.float32)
    z128 = jnp.zeros((N_NODES, 128), jnp.float32)

    p1 = _segsum_sc(x, src, dst, z128)
    h1 = _gin_layer(x, p1[0], p1[1], W1, b1.reshape(1, -1))

    p2 = _segsum_sc(h1, src, dst, z32)
    h2 = _gin_layer(h1, p2[0], p2[1], W2, b2.reshape(1, -1))

    p3 = _segsum_sc(h2, src, dst, z64)
    out = _final_stage(h2, p3[0], p3[1], W3, b3.reshape(1, -1), batch2d,
                       fW1, fb1.reshape(1, -1), fW2, fb2.reshape(1, -1),
                       fW3, fb3.reshape(1, -1))
    return out.reshape(-1)
